# baseline (speedup 1.0000x reference)
import functools

import jax
import jax.numpy as jnp
from jax import lax
from jax.experimental import pallas as pl
from jax.experimental.pallas import tpu as pltpu
from jax.experimental.pallas import tpu_sc as plsc

_EMB_DIM = 64
_NPOS = 16384
_NNEG = 81920
_NC = 2
_NS = 16
_NW = _NC * _NS
_SUB = 128
_NSUB = 2
_CHUNK = _SUB * _NSUB
_POS_PER_W = _NPOS // _NW
_NEG_PER_W = _NNEG // _NW
_POS_PASSES = _POS_PER_W // _CHUNK
_NEG_PASSES = _NEG_PER_W // _CHUNK


def _log_sigmoid(x):
    ax = jnp.abs(x)
    z = jnp.exp(-ax)
    t = z / (z + 2.0)
    t2 = t * t
    p = 1.0 + t2 * (0.33333334 + t2 * (0.2 + t2 * (0.14285715 + t2 * 0.11111111)))
    return jnp.minimum(x, 0.0) - 2.0 * t * p


def _build():
    mesh = plsc.VectorSubcoreMesh(core_axis_name="c", subcore_axis_name="s")

    @functools.partial(
        pl.kernel,
        mesh=mesh,
        compiler_params=pltpu.CompilerParams(
            needs_layout_passes=False, use_tc_tiling_on_sc=True),
        out_type=jax.ShapeDtypeStruct((_NW, 16), jnp.float32),
        scratch_types=[
            pltpu.VMEM((_CHUNK,), jnp.int32),
            pltpu.VMEM((_CHUNK,), jnp.int32),
            pltpu.VMEM((_CHUNK,), jnp.int32),
            pltpu.VMEM((_CHUNK,), jnp.int32),
            pltpu.VMEM((_CHUNK, 2 * _EMB_DIM), jnp.float32),
            pltpu.VMEM((_CHUNK, 2 * _EMB_DIM), jnp.float32),
            pltpu.VMEM((16,), jnp.float32),
            pltpu.SemaphoreType.DMA,
        ],
    )
    def sgns(pw, pc, nw, nc, table2, out,
             idxw_v, idxc_v, hw_v, hc_v, wbuf, cbuf, part_v, sem):
        wid = lax.axis_index("s") * _NC + lax.axis_index("c")
        lane = lax.iota(jnp.int32, 16)

        def run_pass(acc, widx_hbm, cidx_hbm, base, sign):
            pltpu.sync_copy(widx_hbm.at[pl.ds(base, _CHUNK)], idxw_v)
            pltpu.sync_copy(cidx_hbm.at[pl.ds(base, _CHUNK)], idxc_v)

            def halve_body(t, carry):
                iw = idxw_v[pl.ds(t * 16, 16)]
                ic = idxc_v[pl.ds(t * 16, 16)]
                hw_v[pl.ds(t * 16, 16)] = jnp.where(iw >= _HALF, iw - _HALF, iw)
                hc_v[pl.ds(t * 16, 16)] = jnp.where(ic >= _HALF, ic - _HALF, ic)
                return carry
            lax.fori_loop(0, _CHUNK // 16, halve_body, 0)

            copies = []
            for j in range(_NSUB):
                copies.append(pltpu.async_copy(
                    table2.at[hw_v.at[pl.ds(j * _SUB, _SUB)]],
                    wbuf.at[pl.ds(j * _SUB, _SUB)], sem))
                copies.append(pltpu.async_copy(
                    table2.at[hc_v.at[pl.ds(j * _SUB, _SUB)]],
                    cbuf.at[pl.ds(j * _SUB, _SUB)], sem))
            for cp in copies:
                cp.wait()

            def group_body(g, acc):
                parw = (idxw_v[pl.ds(g * 16, 16)] >= _HALF).astype(jnp.int32) * _EMB_DIM
                parc = (idxc_v[pl.ds(g * 16, 16)] >= _HALF).astype(jnp.int32) * _EMB_DIM
                dots = jnp.zeros((16,), jnp.float32)
                for i in range(16):
                    p = g * 16 + i
                    bw = parw[i]
                    bc = parc[i]
                    r = jnp.zeros((16,), jnp.float32)
                    for k in range(_EMB_DIM // 16):
                        wv = wbuf[p, pl.ds(bw + k * 16, 16)]
                        cv = cbuf[p, pl.ds(bc + k * 16, 16)]
                        r = r + wv * cv
                    dots = jnp.where(lane == i, jnp.sum(r), dots)
                return acc + _log_sigmoid(sign * dots)

            return lax.fori_loop(0, _CHUNK // 16, group_body, acc)

        acc = jnp.zeros((16,), jnp.float32)
        for p in range(_POS_PASSES):
            acc = run_pass(acc, pw, pc, wid * _POS_PER_W + p * _CHUNK, 1.0)
        for p in range(_NEG_PASSES):
            acc = run_pass(acc, nw, nc, wid * _NEG_PER_W + p * _CHUNK, -1.0)

        part_v[...] = acc
        pltpu.sync_copy(part_v, out.at[wid])

    return sgns


_HALF = 500224
_RPW = 512


def _repack_body(x1_ref, x2_ref, eye_ref, o_ref):
    eye = eye_ref[...]
    dn = (((0,), (0,)), ((), ()))
    xt1 = lax.dot_general(x1_ref[...], eye, dn,
                          preferred_element_type=jnp.float32)
    xt2 = lax.dot_general(x2_ref[...], eye, dn,
                          preferred_element_type=jnp.float32)
    o_ref[...] = jnp.concatenate([xt1, xt2], axis=1)


def _repack(table_t):
    eye = jnp.eye(_EMB_DIM, dtype=jnp.float32)
    grid = _HALF // _RPW
    return pl.pallas_call(
        _repack_body,
        grid=(grid,),
        in_specs=[
            pl.BlockSpec((_EMB_DIM, _RPW), lambda j: (0, j)),
            pl.BlockSpec((_EMB_DIM, _RPW), lambda j: (0, j + _HALF // _RPW)),
            pl.BlockSpec((_EMB_DIM, _EMB_DIM), lambda j: (0, 0)),
        ],
        out_specs=pl.BlockSpec((_RPW, 2 * _EMB_DIM), lambda j: (j, 0)),
        out_shape=jax.ShapeDtypeStruct((_HALF, 2 * _EMB_DIM), jnp.float32),
    )(table_t, table_t, eye)


_sgns_cache = []


def _get_sgns():
    if not _sgns_cache:
        _sgns_cache.append(_build())
    return _sgns_cache[0]


def kernel(pos_w_idx, pos_c_idx, neg_w_idx, neg_c_idx, W, C):
    pw = pos_w_idx.astype(jnp.int32)
    pc = pos_c_idx.astype(jnp.int32)
    nw = neg_w_idx.astype(jnp.int32)
    nc = neg_c_idx.astype(jnp.int32)
    table2 = _repack(W.T)
    partials = _get_sgns()(pw, pc, nw, nc, table2)
    return -jnp.sum(partials)


# speedup vs baseline: 2.2836x; 2.2836x over previous
import functools

import jax
import jax.numpy as jnp
from jax import lax
from jax.experimental import pallas as pl
from jax.experimental.pallas import tpu as pltpu
from jax.experimental.pallas import tpu_sc as plsc

_EMB_DIM = 64
_NPOS = 16384
_NNEG = 81920
_NC = 2
_NS = 16
_NW = _NC * _NS
_SUB = 128
_NSUB = 2
_CHUNK = _SUB * _NSUB
_POS_PER_W = _NPOS // _NW
_NEG_PER_W = _NNEG // _NW
_POS_PASSES = _POS_PER_W // _CHUNK
_NEG_PASSES = _NEG_PER_W // _CHUNK


def _log_sigmoid(x):
    ax = jnp.abs(x)
    z = jnp.exp(-ax)
    t = z / (z + 2.0)
    t2 = t * t
    p = 1.0 + t2 * (0.33333334 + t2 * (0.2 + t2 * (0.14285715 + t2 * 0.11111111)))
    return jnp.minimum(x, 0.0) - 2.0 * t * p


def _build():
    mesh = plsc.VectorSubcoreMesh(core_axis_name="c", subcore_axis_name="s")

    @functools.partial(
        pl.kernel,
        mesh=mesh,
        compiler_params=pltpu.CompilerParams(
            needs_layout_passes=False, use_tc_tiling_on_sc=True),
        out_type=jax.ShapeDtypeStruct((_NW, 16), jnp.float32),
        scratch_types=[
            pltpu.VMEM((_CHUNK,), jnp.int32),
            pltpu.VMEM((_CHUNK,), jnp.int32),
            pltpu.VMEM((_CHUNK,), jnp.int32),
            pltpu.VMEM((_CHUNK,), jnp.int32),
            pltpu.VMEM((_CHUNK, 2 * _EMB_DIM), jnp.float32),
            pltpu.VMEM((_CHUNK, 2 * _EMB_DIM), jnp.float32),
            pltpu.VMEM((16,), jnp.float32),
            pltpu.SemaphoreType.DMA,
        ],
    )
    def sgns(pw, pc, nw, nc, table2, out,
             idxw_v, idxc_v, hw_v, hc_v, wbuf, cbuf, part_v, sem):
        wid = lax.axis_index("s") * _NC + lax.axis_index("c")
        lane = lax.iota(jnp.int32, 16)

        def run_pass(acc, widx_hbm, cidx_hbm, base, sign):
            pltpu.sync_copy(widx_hbm.at[pl.ds(base, _CHUNK)], idxw_v)
            pltpu.sync_copy(cidx_hbm.at[pl.ds(base, _CHUNK)], idxc_v)

            def halve_body(t, carry):
                iw = idxw_v[pl.ds(t * 16, 16)]
                ic = idxc_v[pl.ds(t * 16, 16)]
                hw_v[pl.ds(t * 16, 16)] = jnp.where(iw >= _HALF, iw - _HALF, iw)
                hc_v[pl.ds(t * 16, 16)] = jnp.where(ic >= _HALF, ic - _HALF, ic)
                return carry
            lax.fori_loop(0, _CHUNK // 16, halve_body, 0)

            copies = []
            for j in range(_NSUB):
                copies.append(pltpu.async_copy(
                    table2.at[hw_v.at[pl.ds(j * _SUB, _SUB)]],
                    wbuf.at[pl.ds(j * _SUB, _SUB)], sem))
                copies.append(pltpu.async_copy(
                    table2.at[hc_v.at[pl.ds(j * _SUB, _SUB)]],
                    cbuf.at[pl.ds(j * _SUB, _SUB)], sem))
            for cp in copies:
                cp.wait()

            def group_body(g, acc):
                parw = (idxw_v[pl.ds(g * 16, 16)] >= _HALF).astype(jnp.int32) * _EMB_DIM
                parc = (idxc_v[pl.ds(g * 16, 16)] >= _HALF).astype(jnp.int32) * _EMB_DIM
                dots = jnp.zeros((16,), jnp.float32)
                for i in range(16):
                    p = g * 16 + i
                    bw = parw[i]
                    bc = parc[i]
                    r = jnp.zeros((16,), jnp.float32)
                    for k in range(_EMB_DIM // 16):
                        wv = wbuf[p, pl.ds(bw + k * 16, 16)]
                        cv = cbuf[p, pl.ds(bc + k * 16, 16)]
                        r = r + wv * cv
                    dots = jnp.where(lane == i, jnp.sum(r), dots)
                return acc + _log_sigmoid(sign * dots)

            return lax.fori_loop(0, _CHUNK // 16, group_body, acc)

        acc = jnp.zeros((16,), jnp.float32)
        for p in range(_POS_PASSES):
            acc = run_pass(acc, pw, pc, wid * _POS_PER_W + p * _CHUNK, 1.0)
        for p in range(_NEG_PASSES):
            acc = run_pass(acc, nw, nc, wid * _NEG_PER_W + p * _CHUNK, -1.0)

        part_v[...] = acc
        pltpu.sync_copy(part_v, out.at[wid])

    return sgns


_HALF = 507904
_RPW = 4096


def _repack_body(x1_ref, x2_ref, eye_ref, o_ref):
    eye = eye_ref[...]
    dn = (((0,), (0,)), ((), ()))
    xt1 = lax.dot_general(x1_ref[...], eye, dn,
                          preferred_element_type=jnp.float32)
    xt2 = lax.dot_general(x2_ref[...], eye, dn,
                          preferred_element_type=jnp.float32)
    o_ref[...] = jnp.concatenate([xt1, xt2], axis=1)


def _repack(table_t):
    eye = jnp.eye(_EMB_DIM, dtype=jnp.float32)
    grid = _HALF // _RPW
    return pl.pallas_call(
        _repack_body,
        grid=(grid,),
        in_specs=[
            pl.BlockSpec((_EMB_DIM, _RPW), lambda j: (0, j)),
            pl.BlockSpec((_EMB_DIM, _RPW),
                         lambda j: (0, jnp.minimum(j + _HALF // _RPW, 244))),
            pl.BlockSpec((_EMB_DIM, _EMB_DIM), lambda j: (0, 0)),
        ],
        out_specs=pl.BlockSpec((_RPW, 2 * _EMB_DIM), lambda j: (j, 0)),
        out_shape=jax.ShapeDtypeStruct((_HALF, 2 * _EMB_DIM), jnp.float32),
    )(table_t, table_t, eye)


_sgns_cache = []


def _get_sgns():
    if not _sgns_cache:
        _sgns_cache.append(_build())
    return _sgns_cache[0]


def kernel(pos_w_idx, pos_c_idx, neg_w_idx, neg_c_idx, W, C):
    pw = pos_w_idx.astype(jnp.int32)
    pc = pos_c_idx.astype(jnp.int32)
    nw = neg_w_idx.astype(jnp.int32)
    nc = neg_c_idx.astype(jnp.int32)
    table2 = _repack(W.T)
    partials = _get_sgns()(pw, pc, nw, nc, table2)
    return -jnp.sum(partials)


# speedup vs baseline: 2.5238x; 1.1052x over previous
import functools

import jax
import jax.numpy as jnp
from jax import lax
from jax.experimental import pallas as pl
from jax.experimental.pallas import tpu as pltpu
from jax.experimental.pallas import tpu_sc as plsc

_EMB_DIM = 64
_NPOS = 16384
_NNEG = 81920
_NC = 2
_NS = 16
_NW = _NC * _NS
_SUB = 128
_NSUB = 2
_CHUNK = _SUB * _NSUB
_POS_PER_W = _NPOS // _NW
_NEG_PER_W = _NNEG // _NW
_POS_PASSES = _POS_PER_W // _CHUNK
_NEG_PASSES = _NEG_PER_W // _CHUNK


def _log_sigmoid(x):
    ax = jnp.abs(x)
    z = jnp.exp(-ax)
    t = z / (z + 2.0)
    t2 = t * t
    p = 1.0 + t2 * (0.33333334 + t2 * (0.2 + t2 * (0.14285715 + t2 * 0.11111111)))
    return jnp.minimum(x, 0.0) - 2.0 * t * p


def _build():
    mesh = plsc.VectorSubcoreMesh(core_axis_name="c", subcore_axis_name="s")

    @functools.partial(
        pl.kernel,
        mesh=mesh,
        compiler_params=pltpu.CompilerParams(
            needs_layout_passes=False, use_tc_tiling_on_sc=True),
        out_type=jax.ShapeDtypeStruct((_NW, 16), jnp.float32),
        scratch_types=[
            pltpu.VMEM((_CHUNK,), jnp.int32),
            pltpu.VMEM((_CHUNK,), jnp.int32),
            pltpu.VMEM((_CHUNK,), jnp.int32),
            pltpu.VMEM((_CHUNK,), jnp.int32),
            pltpu.VMEM((_CHUNK, 2 * _EMB_DIM), jnp.float32),
            pltpu.VMEM((_CHUNK, 2 * _EMB_DIM), jnp.float32),
            pltpu.VMEM((16,), jnp.float32),
            pltpu.SemaphoreType.DMA,
        ],
    )
    def sgns(pw, pc, nw, nc, table2, out,
             idxw_v, idxc_v, hw_v, hc_v, wbuf, cbuf, part_v, sem):
        wid = lax.axis_index("s") * _NC + lax.axis_index("c")
        lane = lax.iota(jnp.int32, 16)

        def run_pass(acc, widx_hbm, cidx_hbm, base, sign):
            pltpu.sync_copy(widx_hbm.at[pl.ds(base, _CHUNK)], idxw_v)
            pltpu.sync_copy(cidx_hbm.at[pl.ds(base, _CHUNK)], idxc_v)

            def halve_body(t, carry):
                iw = idxw_v[pl.ds(t * 16, 16)]
                ic = idxc_v[pl.ds(t * 16, 16)]
                hw_v[pl.ds(t * 16, 16)] = jnp.where(iw >= _HALF, iw - _HALF, iw)
                hc_v[pl.ds(t * 16, 16)] = jnp.where(ic >= _HALF, ic - _HALF, ic)
                return carry
            lax.fori_loop(0, _CHUNK // 16, halve_body, 0)

            copies = []
            for j in range(_NSUB):
                copies.append(pltpu.async_copy(
                    table2.at[hw_v.at[pl.ds(j * _SUB, _SUB)]],
                    wbuf.at[pl.ds(j * _SUB, _SUB)], sem))
                copies.append(pltpu.async_copy(
                    table2.at[hc_v.at[pl.ds(j * _SUB, _SUB)]],
                    cbuf.at[pl.ds(j * _SUB, _SUB)], sem))
            for cp in copies:
                cp.wait()

            def group_body(g, acc):
                parw = (idxw_v[pl.ds(g * 16, 16)] >= _HALF).astype(jnp.int32) * _EMB_DIM
                parc = (idxc_v[pl.ds(g * 16, 16)] >= _HALF).astype(jnp.int32) * _EMB_DIM
                dots = jnp.zeros((16,), jnp.float32)
                for i in range(16):
                    p = g * 16 + i
                    bw = parw[i]
                    bc = parc[i]
                    r = jnp.zeros((16,), jnp.float32)
                    for k in range(_EMB_DIM // 16):
                        wv = wbuf[p, pl.ds(bw + k * 16, 16)]
                        cv = cbuf[p, pl.ds(bc + k * 16, 16)]
                        r = r + wv * cv
                    dots = jnp.where(lane == i, jnp.sum(r), dots)
                return acc + _log_sigmoid(sign * dots)

            return lax.fori_loop(0, _CHUNK // 16, group_body, acc)

        acc = jnp.zeros((16,), jnp.float32)
        for p in range(_POS_PASSES):
            acc = run_pass(acc, pw, pc, wid * _POS_PER_W + p * _CHUNK, 1.0)
        for p in range(_NEG_PASSES):
            acc = run_pass(acc, nw, nc, wid * _NEG_PER_W + p * _CHUNK, -1.0)

        part_v[...] = acc
        pltpu.sync_copy(part_v, out.at[wid])

    return sgns


_HALF = 507904
_RPW = 8192


def _repack_body(x1_ref, x2_ref, eye_ref, o_ref):
    eye = eye_ref[...]
    dn = (((0,), (0,)), ((), ()))
    xt1 = lax.dot_general(x1_ref[...], eye, dn,
                          preferred_element_type=jnp.float32)
    xt2 = lax.dot_general(x2_ref[...], eye, dn,
                          preferred_element_type=jnp.float32)
    o_ref[...] = jnp.concatenate([xt1, xt2], axis=1)


def _repack(table_t):
    eye = jnp.eye(_EMB_DIM, dtype=jnp.float32)
    grid = _HALF // _RPW
    return pl.pallas_call(
        _repack_body,
        grid=(grid,),
        in_specs=[
            pl.BlockSpec((_EMB_DIM, _RPW), lambda j: (0, j)),
            pl.BlockSpec((_EMB_DIM, _RPW),
                         lambda j: (0, jnp.minimum(j + _HALF // _RPW, 122))),
            pl.BlockSpec((_EMB_DIM, _EMB_DIM), lambda j: (0, 0)),
        ],
        out_specs=pl.BlockSpec((_RPW, 2 * _EMB_DIM), lambda j: (j, 0)),
        out_shape=jax.ShapeDtypeStruct((_HALF, 2 * _EMB_DIM), jnp.float32),
    )(table_t, table_t, eye)


_sgns_cache = []


def _get_sgns():
    if not _sgns_cache:
        _sgns_cache.append(_build())
    return _sgns_cache[0]


def kernel(pos_w_idx, pos_c_idx, neg_w_idx, neg_c_idx, W, C):
    pw = pos_w_idx.astype(jnp.int32)
    pc = pos_c_idx.astype(jnp.int32)
    nw = neg_w_idx.astype(jnp.int32)
    nc = neg_c_idx.astype(jnp.int32)
    table2 = _repack(W.T)
    partials = _get_sgns()(pw, pc, nw, nc, table2)
    return -jnp.sum(partials)


# speedup vs baseline: 2.6346x; 1.0439x over previous
import functools

import jax
import jax.numpy as jnp
from jax import lax
from jax.experimental import pallas as pl
from jax.experimental.pallas import tpu as pltpu
from jax.experimental.pallas import tpu_sc as plsc

_EMB_DIM = 64
_NPOS = 16384
_NNEG = 81920
_NC = 2
_NS = 16
_NW = _NC * _NS
_SUB = 128
_NSUB = 2
_CHUNK = _SUB * _NSUB
_POS_PER_W = _NPOS // _NW
_NEG_PER_W = _NNEG // _NW
_POS_PASSES = _POS_PER_W // _CHUNK
_NEG_PASSES = _NEG_PER_W // _CHUNK


def _log_sigmoid(x):
    ax = jnp.abs(x)
    z = jnp.exp(-ax)
    t = z / (z + 2.0)
    t2 = t * t
    p = 1.0 + t2 * (0.33333334 + t2 * (0.2 + t2 * (0.14285715 + t2 * 0.11111111)))
    return jnp.minimum(x, 0.0) - 2.0 * t * p


def _build():
    mesh = plsc.VectorSubcoreMesh(core_axis_name="c", subcore_axis_name="s")

    @functools.partial(
        pl.kernel,
        mesh=mesh,
        compiler_params=pltpu.CompilerParams(
            needs_layout_passes=False, use_tc_tiling_on_sc=True),
        out_type=jax.ShapeDtypeStruct((_NW, 16), jnp.float32),
        scratch_types=[
            pltpu.VMEM((_CHUNK,), jnp.int32),
            pltpu.VMEM((_CHUNK,), jnp.int32),
            pltpu.VMEM((_CHUNK,), jnp.int32),
            pltpu.VMEM((_CHUNK,), jnp.int32),
            pltpu.VMEM((_CHUNK, 2 * _EMB_DIM), jnp.float32),
            pltpu.VMEM((_CHUNK, 2 * _EMB_DIM), jnp.float32),
            pltpu.VMEM((16,), jnp.float32),
            pltpu.SemaphoreType.DMA,
        ],
    )
    def sgns(pw, pc, nw, nc, table2, out,
             idxw_v, idxc_v, hw_v, hc_v, wbuf, cbuf, part_v, sem):
        wid = lax.axis_index("s") * _NC + lax.axis_index("c")
        lane = lax.iota(jnp.int32, 16)

        def run_pass(acc, widx_hbm, cidx_hbm, base, sign):
            pltpu.sync_copy(widx_hbm.at[pl.ds(base, _CHUNK)], idxw_v)
            pltpu.sync_copy(cidx_hbm.at[pl.ds(base, _CHUNK)], idxc_v)

            def halve_body(t, carry):
                iw = idxw_v[pl.ds(t * 16, 16)]
                ic = idxc_v[pl.ds(t * 16, 16)]
                hw_v[pl.ds(t * 16, 16)] = jnp.where(iw >= _HALF, iw - _HALF, iw)
                hc_v[pl.ds(t * 16, 16)] = jnp.where(ic >= _HALF, ic - _HALF, ic)
                return carry
            lax.fori_loop(0, _CHUNK // 16, halve_body, 0)

            copies = []
            for j in range(_NSUB):
                copies.append(pltpu.async_copy(
                    table2.at[hw_v.at[pl.ds(j * _SUB, _SUB)]],
                    wbuf.at[pl.ds(j * _SUB, _SUB)], sem))
                copies.append(pltpu.async_copy(
                    table2.at[hc_v.at[pl.ds(j * _SUB, _SUB)]],
                    cbuf.at[pl.ds(j * _SUB, _SUB)], sem))
            for cp in copies:
                cp.wait()

            def group_body(g, acc):
                parw = (idxw_v[pl.ds(g * 16, 16)] >= _HALF).astype(jnp.int32) * _EMB_DIM
                parc = (idxc_v[pl.ds(g * 16, 16)] >= _HALF).astype(jnp.int32) * _EMB_DIM
                dots = jnp.zeros((16,), jnp.float32)
                for i in range(16):
                    p = g * 16 + i
                    bw = parw[i]
                    bc = parc[i]
                    r = jnp.zeros((16,), jnp.float32)
                    for k in range(_EMB_DIM // 16):
                        wv = wbuf[p, pl.ds(bw + k * 16, 16)]
                        cv = cbuf[p, pl.ds(bc + k * 16, 16)]
                        r = r + wv * cv
                    dots = jnp.where(lane == i, jnp.sum(r), dots)
                return acc + _log_sigmoid(sign * dots)

            return lax.fori_loop(0, _CHUNK // 16, group_body, acc)

        acc = jnp.zeros((16,), jnp.float32)
        for p in range(_POS_PASSES):
            acc = run_pass(acc, pw, pc, wid * _POS_PER_W + p * _CHUNK, 1.0)
        for p in range(_NEG_PASSES):
            acc = run_pass(acc, nw, nc, wid * _NEG_PER_W + p * _CHUNK, -1.0)

        part_v[...] = acc
        pltpu.sync_copy(part_v, out.at[wid])

    return sgns


_HALF = 507904
_RPW = 16384


def _repack_body(x1_ref, x2_ref, eye_ref, o_ref):
    eye = eye_ref[...]
    dn = (((0,), (0,)), ((), ()))
    xt1 = lax.dot_general(x1_ref[...], eye, dn,
                          preferred_element_type=jnp.float32)
    xt2 = lax.dot_general(x2_ref[...], eye, dn,
                          preferred_element_type=jnp.float32)
    o_ref[...] = jnp.concatenate([xt1, xt2], axis=1)


def _repack(table_t):
    eye = jnp.eye(_EMB_DIM, dtype=jnp.float32)
    grid = _HALF // _RPW
    return pl.pallas_call(
        _repack_body,
        grid=(grid,),
        in_specs=[
            pl.BlockSpec((_EMB_DIM, _RPW), lambda j: (0, j)),
            pl.BlockSpec((_EMB_DIM, _RPW),
                         lambda j: (0, jnp.minimum(j + _HALF // _RPW, 61))),
            pl.BlockSpec((_EMB_DIM, _EMB_DIM), lambda j: (0, 0)),
        ],
        out_specs=pl.BlockSpec((_RPW, 2 * _EMB_DIM), lambda j: (j, 0)),
        out_shape=jax.ShapeDtypeStruct((_HALF, 2 * _EMB_DIM), jnp.float32),
    )(table_t, table_t, eye)


_sgns_cache = []


def _get_sgns():
    if not _sgns_cache:
        _sgns_cache.append(_build())
    return _sgns_cache[0]


def kernel(pos_w_idx, pos_c_idx, neg_w_idx, neg_c_idx, W, C):
    pw = pos_w_idx.astype(jnp.int32)
    pc = pos_c_idx.astype(jnp.int32)
    nw = neg_w_idx.astype(jnp.int32)
    nc = neg_c_idx.astype(jnp.int32)
    table2 = _repack(W.T)
    partials = _get_sgns()(pw, pc, nw, nc, table2)
    return -jnp.sum(partials)


# speedup vs baseline: 2.8672x; 1.0883x over previous
import functools

import jax
import jax.numpy as jnp
from jax import lax
from jax.experimental import pallas as pl
from jax.experimental.pallas import tpu as pltpu
from jax.experimental.pallas import tpu_sc as plsc

_EMB_DIM = 64
_NPOS = 16384
_NNEG = 81920
_NC = 2
_NS = 16
_NW = _NC * _NS
_SUB = 128
_NSUB = 2
_CHUNK = _SUB * _NSUB
_POS_PER_W = _NPOS // _NW
_NEG_PER_W = _NNEG // _NW
_POS_PASSES = _POS_PER_W // _CHUNK
_NEG_PASSES = _NEG_PER_W // _CHUNK


def _log_sigmoid(x):
    ax = jnp.abs(x)
    z = jnp.exp(-ax)
    t = z / (z + 2.0)
    t2 = t * t
    p = 1.0 + t2 * (0.33333334 + t2 * (0.2 + t2 * (0.14285715 + t2 * 0.11111111)))
    return jnp.minimum(x, 0.0) - 2.0 * t * p


_PAIRS = _NPOS + _NNEG
_PC = 128
_PER_W = _PAIRS // _NW
_NPASS = _PER_W // _PC


def _build():
    mesh = plsc.VectorSubcoreMesh(core_axis_name="c", subcore_axis_name="s")

    @functools.partial(
        pl.kernel,
        mesh=mesh,
        compiler_params=pltpu.CompilerParams(
            needs_layout_passes=False, use_tc_tiling_on_sc=True),
        out_type=jax.ShapeDtypeStruct((_NW, 16), jnp.float32),
        scratch_types=[
            [pltpu.VMEM((_PC,), jnp.int32)] * 2,
            [pltpu.VMEM((_PC,), jnp.int32)] * 2,
            [pltpu.VMEM((_PC,), jnp.int32)] * 2,
            [pltpu.VMEM((_PC,), jnp.int32)] * 2,
            [pltpu.VMEM((_PC, 2 * _EMB_DIM), jnp.float32)] * 2,
            [pltpu.VMEM((_PC, 2 * _EMB_DIM), jnp.float32)] * 2,
            pltpu.VMEM((16,), jnp.float32),
            [pltpu.SemaphoreType.DMA] * 2,
        ],
    )
    def sgns(wall, call, table2, out,
             idxw, idxc, hw, hc, wb, cb, part_v, sem):
        wid = lax.axis_index("s") * _NC + lax.axis_index("c")
        lane = lax.iota(jnp.int32, 16)
        base0 = wid * _PER_W

        def issue(s, base):
            pltpu.sync_copy(wall.at[pl.ds(base, _PC)], idxw[s])
            pltpu.sync_copy(call.at[pl.ds(base, _PC)], idxc[s])
            for t in range(_PC // 16):
                iw = idxw[s][pl.ds(t * 16, 16)]
                ic = idxc[s][pl.ds(t * 16, 16)]
                hw[s][pl.ds(t * 16, 16)] = jnp.where(iw >= _HALF, iw - _HALF, iw)
                hc[s][pl.ds(t * 16, 16)] = jnp.where(ic >= _HALF, ic - _HALF, ic)
            pltpu.async_copy(table2.at[hw[s]], wb[s], sem[s])
            pltpu.async_copy(table2.at[hc[s]], cb[s], sem[s])

        def drain(s):
            pltpu.make_async_copy(table2.at[hw[s]], wb[s], sem[s]).wait()
            pltpu.make_async_copy(table2.at[hc[s]], cb[s], sem[s]).wait()

        def compute(s, base, acc):
            def group_body(g, acc):
                gidx = base + g * 16 + lane
                sgn = jnp.where(gidx < _NPOS, 1.0, -1.0)
                parw = (idxw[s][pl.ds(g * 16, 16)] >= _HALF).astype(jnp.int32) * _EMB_DIM
                parc = (idxc[s][pl.ds(g * 16, 16)] >= _HALF).astype(jnp.int32) * _EMB_DIM
                dots = jnp.zeros((16,), jnp.float32)
                for i in range(16):
                    p = g * 16 + i
                    bw = parw[i]
                    bc = parc[i]
                    r = jnp.zeros((16,), jnp.float32)
                    for k in range(_EMB_DIM // 16):
                        wv = wb[s][p, pl.ds(bw + k * 16, 16)]
                        cv = cb[s][p, pl.ds(bc + k * 16, 16)]
                        r = r + wv * cv
                    dots = jnp.where(lane == i, jnp.sum(r), dots)
                return acc + _log_sigmoid(sgn * dots)
            return lax.fori_loop(0, _PC // 16, group_body, acc)

        issue(0, base0)

        def two_pass(k, acc):
            base_a = base0 + (2 * k) * _PC
            issue(1, base_a + _PC)
            drain(0)
            acc = compute(0, base_a, acc)

            @pl.when(k < _NPASS // 2 - 1)
            def _():
                issue(0, base_a + 2 * _PC)
            drain(1)
            return compute(1, base_a + _PC, acc)

        acc = lax.fori_loop(0, _NPASS // 2, two_pass,
                            jnp.zeros((16,), jnp.float32))
        part_v[...] = acc
        pltpu.sync_copy(part_v, out.at[wid])

    return sgns


_HALF = 507904
_RPW = 16384


def _repack_body(x1_ref, x2_ref, eye_ref, o_ref):
    eye = eye_ref[...]
    dn = (((0,), (0,)), ((), ()))
    xt1 = lax.dot_general(x1_ref[...], eye, dn,
                          preferred_element_type=jnp.float32)
    xt2 = lax.dot_general(x2_ref[...], eye, dn,
                          preferred_element_type=jnp.float32)
    o_ref[...] = jnp.concatenate([xt1, xt2], axis=1)


def _repack(table_t):
    eye = jnp.eye(_EMB_DIM, dtype=jnp.float32)
    grid = _HALF // _RPW
    return pl.pallas_call(
        _repack_body,
        grid=(grid,),
        in_specs=[
            pl.BlockSpec((_EMB_DIM, _RPW), lambda j: (0, j)),
            pl.BlockSpec((_EMB_DIM, _RPW),
                         lambda j: (0, jnp.minimum(j + _HALF // _RPW, 61))),
            pl.BlockSpec((_EMB_DIM, _EMB_DIM), lambda j: (0, 0)),
        ],
        out_specs=pl.BlockSpec((_RPW, 2 * _EMB_DIM), lambda j: (j, 0)),
        out_shape=jax.ShapeDtypeStruct((_HALF, 2 * _EMB_DIM), jnp.float32),
    )(table_t, table_t, eye)


_sgns_cache = []


def _get_sgns():
    if not _sgns_cache:
        _sgns_cache.append(_build())
    return _sgns_cache[0]


def kernel(pos_w_idx, pos_c_idx, neg_w_idx, neg_c_idx, W, C):
    wall = jnp.concatenate([pos_w_idx.astype(jnp.int32),
                            neg_w_idx.astype(jnp.int32)])
    call = jnp.concatenate([pos_c_idx.astype(jnp.int32),
                            neg_c_idx.astype(jnp.int32)])
    table2 = _repack(W.T)
    partials = _get_sgns()(wall, call, table2)
    return -jnp.sum(partials)
